# baseline (speedup 1.0000x reference)
import functools

import jax
import jax.numpy as jnp
from jax import lax
from jax.experimental import pallas as pl
from jax.experimental.pallas import tpu as pltpu
from jax.experimental.pallas import tpu_sc as plsc

DIM = 32
B = 16384
MARGIN = 1.0
C = 0.25

NC = 2
NS = 16
NW = NC * NS

EB = 4 * B // NW
RB = 2 * B // NW

_TR_BLK = 2048


def _tr_body(xt_ref, out_ref):
    xt = xt_ref[...]
    eye = jax.lax.broadcasted_iota(jnp.int32, (DIM, DIM), 0) == \
        jax.lax.broadcasted_iota(jnp.int32, (DIM, DIM), 1)
    out_ref[...] = jax.lax.dot_general(
        xt, eye.astype(jnp.float32), (((0,), (0,)), ((), ())),
        preferred_element_type=jnp.float32)


def _transpose_table(table):
    n = table.shape[0]
    tt = table.T
    grid = (n + _TR_BLK - 1) // _TR_BLK
    return pl.pallas_call(
        _tr_body,
        grid=(grid,),
        in_specs=[pl.BlockSpec((DIM, _TR_BLK), lambda c: (0, c))],
        out_specs=pl.BlockSpec((_TR_BLK, DIM), lambda c: (c, 0)),
        out_shape=jax.ShapeDtypeStruct((n, DIM), jnp.float32),
    )(tt)


def _sc_gather(ent_emb, ent_idx, rel_emb, rel_idx):
    mesh = plsc.VectorSubcoreMesh(core_axis_name="c", subcore_axis_name="s")

    @functools.partial(
        pl.kernel,
        out_type=(
            jax.ShapeDtypeStruct((4 * B, DIM), jnp.float32),
            jax.ShapeDtypeStruct((2 * B, DIM), jnp.float32),
        ),
        mesh=mesh,
        scratch_types=[
            pltpu.VMEM((EB,), jnp.int32),
            pltpu.VMEM((EB, DIM), jnp.float32),
            pltpu.VMEM((RB,), jnp.int32),
            pltpu.VMEM((RB, DIM), jnp.float32),
            pltpu.SemaphoreType.DMA,
        ],
        compiler_params=pltpu.CompilerParams(use_tc_tiling_on_sc=False),
    )
    def k(ent_hbm, eidx_hbm, rel_hbm, ridx_hbm, ent_out, rel_out,
          eidx_v, erows_v, ridx_v, rrows_v, sem):
        wid = lax.axis_index("s") * NC + lax.axis_index("c")
        eb = wid * EB
        rb = wid * RB
        pltpu.sync_copy(eidx_hbm.at[pl.ds(eb, EB)], eidx_v)
        pltpu.sync_copy(ridx_hbm.at[pl.ds(rb, RB)], ridx_v)
        ec = pltpu.async_copy(ent_hbm.at[eidx_v], erows_v, sem)
        rc = pltpu.async_copy(rel_hbm.at[ridx_v], rrows_v, sem)
        ec.wait()
        rc.wait()
        pltpu.sync_copy(erows_v, ent_out.at[pl.ds(eb, EB)])
        pltpu.sync_copy(rrows_v, rel_out.at[pl.ds(rb, RB)])

    return k(ent_emb, ent_idx, rel_emb, rel_idx)


_TC_CHUNK = 2048


def _tc_body(h_ref, r_ref, t_ref, nh_ref, nr_ref, nt_ref, out_ref):
    h = h_ref[...]
    r = r_ref[...]
    t = t_ref[...]
    nh = nh_ref[...]
    nr = nr_ref[...]
    nt = nt_ref[...]

    pd = h + r - t
    nd = nh + nr - nt
    psq = jnp.sum(pd * pd, axis=1, keepdims=True)
    nsq = jnp.sum(nd * nd, axis=1, keepdims=True)
    marg = jnp.maximum(jnp.sqrt(psq) - jnp.sqrt(nsq) + MARGIN, 0.0)

    def rowreg(x):
        return jnp.maximum(jnp.sum(x * x, axis=1, keepdims=True) - 1.0, 0.0)

    ereg = rowreg(h) + rowreg(t) + rowreg(nh) + rowreg(nt)
    rreg = rowreg(r) + rowreg(nr)

    val = (jnp.sum(marg) / B
           + C * (jnp.sum(ereg) / (4 * B) + jnp.sum(rreg) / (2 * B)))

    @pl.when(pl.program_id(0) == 0)
    def _():
        out_ref[0, 0] = 0.0

    out_ref[0, 0] += val


def _tc_loss(ent_rows, rel_rows):
    grid = B // _TC_CHUNK
    blk = (_TC_CHUNK, DIM)

    def espec(region):
        return pl.BlockSpec(blk, lambda c, region=region: (region * grid + c, 0))

    out = pl.pallas_call(
        _tc_body,
        grid=(grid,),
        in_specs=[
            espec(0),
            pl.BlockSpec(blk, lambda c: (c, 0)),
            espec(1),
            espec(2),
            pl.BlockSpec(blk, lambda c: (grid + c, 0)),
            espec(3),
        ],
        out_specs=pl.BlockSpec(
            (1, 1), lambda c: (0, 0), memory_space=pltpu.SMEM),
        out_shape=jax.ShapeDtypeStruct((1, 1), jnp.float32),
    )(ent_rows, rel_rows, ent_rows, ent_rows, rel_rows, ent_rows)
    return out


def kernel(current_triples, corrupted_triples, ent_emb_1, rel_emb_1):
    ent_idx = jnp.concatenate([
        current_triples[:, 0], current_triples[:, 2],
        corrupted_triples[:, 0], corrupted_triples[:, 2],
    ])
    rel_idx = jnp.concatenate([current_triples[:, 1], corrupted_triples[:, 1]])
    ent_packed = _transpose_table(ent_emb_1)
    rel_packed = _transpose_table(rel_emb_1)
    ent_rows, rel_rows = _sc_gather(ent_packed, ent_idx, rel_packed, rel_idx)
    out = _tc_loss(ent_rows, rel_rows)
    return jnp.reshape(out, ())


# speedup vs baseline: 3.0568x; 3.0568x over previous
import functools

import jax
import jax.numpy as jnp
from jax import lax
from jax.experimental import pallas as pl
from jax.experimental.pallas import tpu as pltpu
from jax.experimental.pallas import tpu_sc as plsc

DIM = 32
PACK = 4
B = 16384
MARGIN = 1.0
C = 0.25

NC = 2
NS = 16
NW = NC * NS

EB = 4 * B // NW
RB = 2 * B // NW
CH = 256

_W = 2048


def _repack_body(b0_ref, b1_ref, b2_ref, b3_ref, out_ref):
    x = jnp.concatenate(
        [b0_ref[...], b1_ref[...], b2_ref[...], b3_ref[...]], axis=0)
    eye = (jax.lax.broadcasted_iota(jnp.int32, (PACK * DIM, PACK * DIM), 0)
           == jax.lax.broadcasted_iota(jnp.int32, (PACK * DIM, PACK * DIM), 1))
    out_ref[...] = jax.lax.dot_general(
        x, eye.astype(jnp.float32), (((0,), (0,)), ((), ())),
        preferred_element_type=jnp.float32)


def _repack_table(table):
    n = table.shape[0]
    tt = table.T
    grid = -(-n // (PACK * _W))
    last = -(-n // _W) - 1

    def sub(j):
        return pl.BlockSpec(
            (DIM, _W), lambda c, j=j: (0, jnp.minimum(PACK * c + j, last)))

    return pl.pallas_call(
        _repack_body,
        grid=(grid,),
        in_specs=[sub(0), sub(1), sub(2), sub(3)],
        out_specs=pl.BlockSpec((_W, PACK * DIM), lambda c: (c, 0)),
        out_shape=jax.ShapeDtypeStruct((grid * _W, PACK * DIM), jnp.float32),
    )(tt, tt, tt, tt)


def _packed_pos(idx):
    c = idx // (PACK * _W)
    j = (idx % (PACK * _W)) // _W
    m = idx % _W
    return c * _W + m, j


def _sc_gather(ent_emb, ent_gidx, rel_emb, rel_gidx):
    mesh = plsc.VectorSubcoreMesh(core_axis_name="c", subcore_axis_name="s")

    @functools.partial(
        pl.kernel,
        out_type=(
            jax.ShapeDtypeStruct((4 * B, PACK * DIM), jnp.float32),
            jax.ShapeDtypeStruct((2 * B, PACK * DIM), jnp.float32),
        ),
        mesh=mesh,
        scratch_types=[
            pltpu.VMEM((EB,), jnp.int32),
            pltpu.VMEM((RB,), jnp.int32),
            pltpu.VMEM((CH, PACK * DIM), jnp.float32),
            pltpu.VMEM((CH, PACK * DIM), jnp.float32),
            pltpu.SemaphoreType.DMA,
            pltpu.SemaphoreType.DMA,
        ],
        compiler_params=pltpu.CompilerParams(use_tc_tiling_on_sc=False),
    )
    def k(ent_hbm, eidx_hbm, rel_hbm, ridx_hbm, ent_out, rel_out,
          eidx_v, ridx_v, buf0, buf1, sem0, sem1):
        wid = lax.axis_index("s") * NC + lax.axis_index("c")
        eb = wid * EB
        rb = wid * RB
        pltpu.sync_copy(eidx_hbm.at[pl.ds(eb, EB)], eidx_v)
        pltpu.sync_copy(ridx_hbm.at[pl.ds(rb, RB)], ridx_v)

        chunks = [(ent_hbm, eidx_v, j * CH, ent_out, eb)
                  for j in range(EB // CH)]
        chunks += [(rel_hbm, ridx_v, j * CH, rel_out, rb)
                   for j in range(RB // CH)]

        bufs = (buf0, buf1)
        sems = (sem0, sem1)
        n = len(chunks)
        copies = [None] * n
        for i in range(n + 1):
            if i < n:
                tab, idx, off, _, _ = chunks[i]
                copies[i] = pltpu.async_copy(
                    tab.at[idx.at[pl.ds(off, CH)]], bufs[i % 2], sems[i % 2])
            if i >= 1:
                _, _, off, out, base = chunks[i - 1]
                copies[i - 1].wait()
                pltpu.sync_copy(bufs[(i - 1) % 2],
                                out.at[pl.ds(base + off, CH)])

    return k(ent_emb, ent_gidx, rel_emb, rel_gidx)


_TC_CHUNK = 2048


def _select_row(x, o):
    acc = jnp.where(o == 0, x[:, 0:DIM], 0.0)
    for k in range(1, PACK):
        acc += jnp.where(o == k, x[:, k * DIM:(k + 1) * DIM], 0.0)
    return acc


def _tc_body(h_ref, r_ref, t_ref, nh_ref, nr_ref, nt_ref,
             oh_ref, or_ref, ot_ref, onh_ref, onr_ref, ont_ref, out_ref):
    h = _select_row(h_ref[...], oh_ref[...])
    r = _select_row(r_ref[...], or_ref[...])
    t = _select_row(t_ref[...], ot_ref[...])
    nh = _select_row(nh_ref[...], onh_ref[...])
    nr = _select_row(nr_ref[...], onr_ref[...])
    nt = _select_row(nt_ref[...], ont_ref[...])

    pd = h + r - t
    nd = nh + nr - nt
    psq = jnp.sum(pd * pd, axis=1, keepdims=True)
    nsq = jnp.sum(nd * nd, axis=1, keepdims=True)
    marg = jnp.maximum(jnp.sqrt(psq) - jnp.sqrt(nsq) + MARGIN, 0.0)

    def rowreg(x):
        return jnp.maximum(jnp.sum(x * x, axis=1, keepdims=True) - 1.0, 0.0)

    ereg = rowreg(h) + rowreg(t) + rowreg(nh) + rowreg(nt)
    rreg = rowreg(r) + rowreg(nr)

    val = (jnp.sum(marg) / B
           + C * (jnp.sum(ereg) / (4 * B) + jnp.sum(rreg) / (2 * B)))

    @pl.when(pl.program_id(0) == 0)
    def _():
        out_ref[0, 0] = 0.0

    out_ref[0, 0] += val


def _tc_loss(ent_rows, rel_rows, ent_off, rel_off):
    grid = B // _TC_CHUNK
    blk = (_TC_CHUNK, PACK * DIM)
    oblk = (_TC_CHUNK, 1)

    def espec(region, b):
        return pl.BlockSpec(b, lambda c, region=region: (region * grid + c, 0))

    out = pl.pallas_call(
        _tc_body,
        grid=(grid,),
        in_specs=[
            espec(0, blk),
            pl.BlockSpec(blk, lambda c: (c, 0)),
            espec(1, blk),
            espec(2, blk),
            pl.BlockSpec(blk, lambda c: (grid + c, 0)),
            espec(3, blk),
            espec(0, oblk),
            pl.BlockSpec(oblk, lambda c: (c, 0)),
            espec(1, oblk),
            espec(2, oblk),
            pl.BlockSpec(oblk, lambda c: (grid + c, 0)),
            espec(3, oblk),
        ],
        out_specs=pl.BlockSpec(
            (1, 1), lambda c: (0, 0), memory_space=pltpu.SMEM),
        out_shape=jax.ShapeDtypeStruct((1, 1), jnp.float32),
    )(ent_rows, rel_rows, ent_rows, ent_rows, rel_rows, ent_rows,
      ent_off, rel_off, ent_off, ent_off, rel_off, ent_off)
    return out


def kernel(current_triples, corrupted_triples, ent_emb_1, rel_emb_1):
    ent_idx = jnp.concatenate([
        current_triples[:, 0], current_triples[:, 2],
        corrupted_triples[:, 0], corrupted_triples[:, 2],
    ])
    rel_idx = jnp.concatenate([current_triples[:, 1], corrupted_triples[:, 1]])

    ent_packed = _repack_table(ent_emb_1)
    rel_packed = _repack_table(rel_emb_1)
    ent_gidx, ent_o = _packed_pos(ent_idx)
    rel_gidx, rel_o = _packed_pos(rel_idx)
    ent_off = ent_o.reshape(-1, 1)
    rel_off = rel_o.reshape(-1, 1)

    ent_rows, rel_rows = _sc_gather(ent_packed, ent_gidx, rel_packed, rel_gidx)
    out = _tc_loss(ent_rows, rel_rows, ent_off, rel_off)
    return jnp.reshape(out, ())


# speedup vs baseline: 3.5739x; 1.1692x over previous
import functools

import jax
import jax.numpy as jnp
from jax import lax
from jax.experimental import pallas as pl
from jax.experimental.pallas import tpu as pltpu
from jax.experimental.pallas import tpu_sc as plsc

DIM = 32
PACK = 4
B = 16384
MARGIN = 1.0
C = 0.25

NC = 2
NS = 16
NW = NC * NS

EB = 4 * B // NW
RB = 2 * B // NW
CH = 256

_W = 4096


def _repack_body(b0_ref, b1_ref, b2_ref, b3_ref, out_ref):
    x = jnp.concatenate(
        [b0_ref[...], b1_ref[...], b2_ref[...], b3_ref[...]], axis=0)
    eye = (jax.lax.broadcasted_iota(jnp.int32, (PACK * DIM, PACK * DIM), 0)
           == jax.lax.broadcasted_iota(jnp.int32, (PACK * DIM, PACK * DIM), 1))
    out_ref[...] = jax.lax.dot_general(
        x, eye.astype(jnp.float32), (((0,), (0,)), ((), ())),
        preferred_element_type=jnp.float32)


def _repack_table(table):
    n = table.shape[0]
    tt = table.T
    grid = -(-n // (PACK * _W))
    last = -(-n // _W) - 1

    def sub(j):
        return pl.BlockSpec(
            (DIM, _W), lambda c, j=j: (0, jnp.minimum(PACK * c + j, last)))

    return pl.pallas_call(
        _repack_body,
        grid=(grid,),
        in_specs=[sub(0), sub(1), sub(2), sub(3)],
        out_specs=pl.BlockSpec((_W, PACK * DIM), lambda c: (c, 0)),
        out_shape=jax.ShapeDtypeStruct((grid * _W, PACK * DIM), jnp.float32),
    )(tt, tt, tt, tt)


def _packed_pos(idx):
    c = idx // (PACK * _W)
    j = (idx % (PACK * _W)) // _W
    m = idx % _W
    return c * _W + m, j


def _sc_gather(ent_emb, ent_gidx, rel_emb, rel_gidx):
    mesh = plsc.VectorSubcoreMesh(core_axis_name="c", subcore_axis_name="s")

    @functools.partial(
        pl.kernel,
        out_type=(
            jax.ShapeDtypeStruct((4 * B, PACK * DIM), jnp.float32),
            jax.ShapeDtypeStruct((2 * B, PACK * DIM), jnp.float32),
        ),
        mesh=mesh,
        scratch_types=[
            pltpu.VMEM((EB,), jnp.int32),
            pltpu.VMEM((RB,), jnp.int32),
            pltpu.VMEM((CH, PACK * DIM), jnp.float32),
            pltpu.VMEM((CH, PACK * DIM), jnp.float32),
            pltpu.SemaphoreType.DMA,
            pltpu.SemaphoreType.DMA,
        ],
        compiler_params=pltpu.CompilerParams(use_tc_tiling_on_sc=False),
    )
    def k(ent_hbm, eidx_hbm, rel_hbm, ridx_hbm, ent_out, rel_out,
          eidx_v, ridx_v, buf0, buf1, sem0, sem1):
        wid = lax.axis_index("s") * NC + lax.axis_index("c")
        eb = wid * EB
        rb = wid * RB
        pltpu.sync_copy(eidx_hbm.at[pl.ds(eb, EB)], eidx_v)
        pltpu.sync_copy(ridx_hbm.at[pl.ds(rb, RB)], ridx_v)

        chunks = [(ent_hbm, eidx_v, j * CH, ent_out, eb)
                  for j in range(EB // CH)]
        chunks += [(rel_hbm, ridx_v, j * CH, rel_out, rb)
                   for j in range(RB // CH)]

        bufs = (buf0, buf1)
        sems = (sem0, sem1)
        n = len(chunks)
        copies = [None] * n
        for i in range(n + 1):
            if i < n:
                tab, idx, off, _, _ = chunks[i]
                copies[i] = pltpu.async_copy(
                    tab.at[idx.at[pl.ds(off, CH)]], bufs[i % 2], sems[i % 2])
            if i >= 1:
                _, _, off, out, base = chunks[i - 1]
                copies[i - 1].wait()
                pltpu.sync_copy(bufs[(i - 1) % 2],
                                out.at[pl.ds(base + off, CH)])

    return k(ent_emb, ent_gidx, rel_emb, rel_gidx)


_TC_CHUNK = 2048


def _select_row(x, o):
    acc = jnp.where(o == 0, x[:, 0:DIM], 0.0)
    for k in range(1, PACK):
        acc += jnp.where(o == k, x[:, k * DIM:(k + 1) * DIM], 0.0)
    return acc


def _tc_body(h_ref, r_ref, t_ref, nh_ref, nr_ref, nt_ref,
             oh_ref, or_ref, ot_ref, onh_ref, onr_ref, ont_ref, out_ref):
    h = _select_row(h_ref[...], oh_ref[...])
    r = _select_row(r_ref[...], or_ref[...])
    t = _select_row(t_ref[...], ot_ref[...])
    nh = _select_row(nh_ref[...], onh_ref[...])
    nr = _select_row(nr_ref[...], onr_ref[...])
    nt = _select_row(nt_ref[...], ont_ref[...])

    pd = h + r - t
    nd = nh + nr - nt
    psq = jnp.sum(pd * pd, axis=1, keepdims=True)
    nsq = jnp.sum(nd * nd, axis=1, keepdims=True)
    marg = jnp.maximum(jnp.sqrt(psq) - jnp.sqrt(nsq) + MARGIN, 0.0)

    def rowreg(x):
        return jnp.maximum(jnp.sum(x * x, axis=1, keepdims=True) - 1.0, 0.0)

    ereg = rowreg(h) + rowreg(t) + rowreg(nh) + rowreg(nt)
    rreg = rowreg(r) + rowreg(nr)

    val = (jnp.sum(marg) / B
           + C * (jnp.sum(ereg) / (4 * B) + jnp.sum(rreg) / (2 * B)))

    @pl.when(pl.program_id(0) == 0)
    def _():
        out_ref[0, 0] = 0.0

    out_ref[0, 0] += val


def _tc_loss(ent_rows, rel_rows, ent_off, rel_off):
    grid = B // _TC_CHUNK
    blk = (_TC_CHUNK, PACK * DIM)
    oblk = (_TC_CHUNK, 1)

    def espec(region, b):
        return pl.BlockSpec(b, lambda c, region=region: (region * grid + c, 0))

    out = pl.pallas_call(
        _tc_body,
        grid=(grid,),
        in_specs=[
            espec(0, blk),
            pl.BlockSpec(blk, lambda c: (c, 0)),
            espec(1, blk),
            espec(2, blk),
            pl.BlockSpec(blk, lambda c: (grid + c, 0)),
            espec(3, blk),
            espec(0, oblk),
            pl.BlockSpec(oblk, lambda c: (c, 0)),
            espec(1, oblk),
            espec(2, oblk),
            pl.BlockSpec(oblk, lambda c: (grid + c, 0)),
            espec(3, oblk),
        ],
        out_specs=pl.BlockSpec(
            (1, 1), lambda c: (0, 0), memory_space=pltpu.SMEM),
        out_shape=jax.ShapeDtypeStruct((1, 1), jnp.float32),
    )(ent_rows, rel_rows, ent_rows, ent_rows, rel_rows, ent_rows,
      ent_off, rel_off, ent_off, ent_off, rel_off, ent_off)
    return out


def kernel(current_triples, corrupted_triples, ent_emb_1, rel_emb_1):
    ent_idx = jnp.concatenate([
        current_triples[:, 0], current_triples[:, 2],
        corrupted_triples[:, 0], corrupted_triples[:, 2],
    ])
    rel_idx = jnp.concatenate([current_triples[:, 1], corrupted_triples[:, 1]])

    ent_packed = _repack_table(ent_emb_1)
    rel_packed = _repack_table(rel_emb_1)
    ent_gidx, ent_o = _packed_pos(ent_idx)
    rel_gidx, rel_o = _packed_pos(rel_idx)
    ent_off = ent_o.reshape(-1, 1)
    rel_off = rel_o.reshape(-1, 1)

    ent_rows, rel_rows = _sc_gather(ent_packed, ent_gidx, rel_packed, rel_gidx)
    out = _tc_loss(ent_rows, rel_rows, ent_off, rel_off)
    return jnp.reshape(out, ())


# speedup vs baseline: 3.8261x; 1.0706x over previous
import functools

import jax
import jax.numpy as jnp
from jax import lax
from jax.experimental import pallas as pl
from jax.experimental.pallas import tpu as pltpu
from jax.experimental.pallas import tpu_sc as plsc

DIM = 32
PACK = 4
B = 16384
MARGIN = 1.0
C = 0.25

NC = 2
NS = 16
NW = NC * NS

EB = 4 * B // NW
RB = 2 * B // NW
CH = 256

_W = 8192


def _repack_body(b0_ref, b1_ref, b2_ref, b3_ref, out_ref):
    x = jnp.concatenate(
        [b0_ref[...], b1_ref[...], b2_ref[...], b3_ref[...]], axis=0)
    eye = (jax.lax.broadcasted_iota(jnp.int32, (PACK * DIM, PACK * DIM), 0)
           == jax.lax.broadcasted_iota(jnp.int32, (PACK * DIM, PACK * DIM), 1))
    out_ref[...] = jax.lax.dot_general(
        x, eye.astype(jnp.float32), (((0,), (0,)), ((), ())),
        preferred_element_type=jnp.float32)


def _repack_table(table):
    n = table.shape[0]
    tt = table.T
    grid = -(-n // (PACK * _W))
    last = -(-n // _W) - 1

    def sub(j):
        return pl.BlockSpec(
            (DIM, _W), lambda c, j=j: (0, jnp.minimum(PACK * c + j, last)))

    return pl.pallas_call(
        _repack_body,
        grid=(grid,),
        in_specs=[sub(0), sub(1), sub(2), sub(3)],
        out_specs=pl.BlockSpec((_W, PACK * DIM), lambda c: (c, 0)),
        out_shape=jax.ShapeDtypeStruct((grid * _W, PACK * DIM), jnp.float32),
    )(tt, tt, tt, tt)


def _packed_pos(idx):
    c = idx // (PACK * _W)
    j = (idx % (PACK * _W)) // _W
    m = idx % _W
    return c * _W + m, j


def _sc_gather(ent_emb, ent_gidx, rel_emb, rel_gidx):
    mesh = plsc.VectorSubcoreMesh(core_axis_name="c", subcore_axis_name="s")

    @functools.partial(
        pl.kernel,
        out_type=(
            jax.ShapeDtypeStruct((4 * B, PACK * DIM), jnp.float32),
            jax.ShapeDtypeStruct((2 * B, PACK * DIM), jnp.float32),
        ),
        mesh=mesh,
        scratch_types=[
            pltpu.VMEM((EB,), jnp.int32),
            pltpu.VMEM((RB,), jnp.int32),
            pltpu.VMEM((CH, PACK * DIM), jnp.float32),
            pltpu.VMEM((CH, PACK * DIM), jnp.float32),
            pltpu.SemaphoreType.DMA,
            pltpu.SemaphoreType.DMA,
        ],
        compiler_params=pltpu.CompilerParams(use_tc_tiling_on_sc=False),
    )
    def k(ent_hbm, eidx_hbm, rel_hbm, ridx_hbm, ent_out, rel_out,
          eidx_v, ridx_v, buf0, buf1, sem0, sem1):
        wid = lax.axis_index("s") * NC + lax.axis_index("c")
        eb = wid * EB
        rb = wid * RB
        pltpu.sync_copy(eidx_hbm.at[pl.ds(eb, EB)], eidx_v)
        pltpu.sync_copy(ridx_hbm.at[pl.ds(rb, RB)], ridx_v)

        chunks = [(ent_hbm, eidx_v, j * CH, ent_out, eb)
                  for j in range(EB // CH)]
        chunks += [(rel_hbm, ridx_v, j * CH, rel_out, rb)
                   for j in range(RB // CH)]

        bufs = (buf0, buf1)
        sems = (sem0, sem1)
        n = len(chunks)
        copies = [None] * n
        for i in range(n + 1):
            if i < n:
                tab, idx, off, _, _ = chunks[i]
                copies[i] = pltpu.async_copy(
                    tab.at[idx.at[pl.ds(off, CH)]], bufs[i % 2], sems[i % 2])
            if i >= 1:
                _, _, off, out, base = chunks[i - 1]
                copies[i - 1].wait()
                pltpu.sync_copy(bufs[(i - 1) % 2],
                                out.at[pl.ds(base + off, CH)])

    return k(ent_emb, ent_gidx, rel_emb, rel_gidx)


_TC_CHUNK = 2048


def _select_row(x, o):
    acc = jnp.where(o == 0, x[:, 0:DIM], 0.0)
    for k in range(1, PACK):
        acc += jnp.where(o == k, x[:, k * DIM:(k + 1) * DIM], 0.0)
    return acc


def _tc_body(h_ref, r_ref, t_ref, nh_ref, nr_ref, nt_ref,
             oh_ref, or_ref, ot_ref, onh_ref, onr_ref, ont_ref, out_ref):
    h = _select_row(h_ref[...], oh_ref[...])
    r = _select_row(r_ref[...], or_ref[...])
    t = _select_row(t_ref[...], ot_ref[...])
    nh = _select_row(nh_ref[...], onh_ref[...])
    nr = _select_row(nr_ref[...], onr_ref[...])
    nt = _select_row(nt_ref[...], ont_ref[...])

    pd = h + r - t
    nd = nh + nr - nt
    psq = jnp.sum(pd * pd, axis=1, keepdims=True)
    nsq = jnp.sum(nd * nd, axis=1, keepdims=True)
    marg = jnp.maximum(jnp.sqrt(psq) - jnp.sqrt(nsq) + MARGIN, 0.0)

    def rowreg(x):
        return jnp.maximum(jnp.sum(x * x, axis=1, keepdims=True) - 1.0, 0.0)

    ereg = rowreg(h) + rowreg(t) + rowreg(nh) + rowreg(nt)
    rreg = rowreg(r) + rowreg(nr)

    val = (jnp.sum(marg) / B
           + C * (jnp.sum(ereg) / (4 * B) + jnp.sum(rreg) / (2 * B)))

    @pl.when(pl.program_id(0) == 0)
    def _():
        out_ref[0, 0] = 0.0

    out_ref[0, 0] += val


def _tc_loss(ent_rows, rel_rows, ent_off, rel_off):
    grid = B // _TC_CHUNK
    blk = (_TC_CHUNK, PACK * DIM)
    oblk = (_TC_CHUNK, 1)

    def espec(region, b):
        return pl.BlockSpec(b, lambda c, region=region: (region * grid + c, 0))

    out = pl.pallas_call(
        _tc_body,
        grid=(grid,),
        in_specs=[
            espec(0, blk),
            pl.BlockSpec(blk, lambda c: (c, 0)),
            espec(1, blk),
            espec(2, blk),
            pl.BlockSpec(blk, lambda c: (grid + c, 0)),
            espec(3, blk),
            espec(0, oblk),
            pl.BlockSpec(oblk, lambda c: (c, 0)),
            espec(1, oblk),
            espec(2, oblk),
            pl.BlockSpec(oblk, lambda c: (grid + c, 0)),
            espec(3, oblk),
        ],
        out_specs=pl.BlockSpec(
            (1, 1), lambda c: (0, 0), memory_space=pltpu.SMEM),
        out_shape=jax.ShapeDtypeStruct((1, 1), jnp.float32),
    )(ent_rows, rel_rows, ent_rows, ent_rows, rel_rows, ent_rows,
      ent_off, rel_off, ent_off, ent_off, rel_off, ent_off)
    return out


def kernel(current_triples, corrupted_triples, ent_emb_1, rel_emb_1):
    ent_idx = jnp.concatenate([
        current_triples[:, 0], current_triples[:, 2],
        corrupted_triples[:, 0], corrupted_triples[:, 2],
    ])
    rel_idx = jnp.concatenate([current_triples[:, 1], corrupted_triples[:, 1]])

    ent_packed = _repack_table(ent_emb_1)
    rel_packed = _repack_table(rel_emb_1)
    ent_gidx, ent_o = _packed_pos(ent_idx)
    rel_gidx, rel_o = _packed_pos(rel_idx)
    ent_off = ent_o.reshape(-1, 1)
    rel_off = rel_o.reshape(-1, 1)

    ent_rows, rel_rows = _sc_gather(ent_packed, ent_gidx, rel_packed, rel_gidx)
    out = _tc_loss(ent_rows, rel_rows, ent_off, rel_off)
    return jnp.reshape(out, ())
